# Initial kernel scaffold; baseline (speedup 1.0000x reference)
import jax
import jax.numpy as jnp
from jax.experimental import pallas as pl


def kernel(ent_emb, history_vector, candidate_rel_ids, candidate_to_ids, rel_table, ent_table, W1, b1, W2, b2):
    raise NotImplementedError("write your pallas kernel here")


# speedup vs baseline: 4.1993x; 4.1993x over previous
import dataclasses
import functools

import jax
import jax.numpy as jnp
from jax import lax
from jax.experimental import pallas as pl
from jax.experimental.pallas import tpu as pltpu
from jax.experimental.pallas import tpu_sc as plsc

EMB = 128
N_CAND = 100000
N_ENT = 100000
N_REL = 1000
REL_PAD = 1024

NC, NS = 2, 16
NW = NC * NS
CHUNK = 3136
NPAD = NW * CHUNK

ENT_BLK = 2000


def _prep_body(ent_ref, hist_ref, w1_ref, b1_ref, w2_ref, b2_ref, rel_ref,
               rel_o, f2_o):
    feat = jnp.concatenate([ent_ref[...], hist_ref[...]], axis=1)
    h = jnp.maximum(
        jnp.dot(feat, w1_ref[...], preferred_element_type=jnp.float32)
        + b1_ref[...], 0.0)
    feature = jnp.maximum(
        jnp.dot(h, w2_ref[...], preferred_element_type=jnp.float32)
        + b2_ref[...], 0.0)
    r = lax.broadcasted_iota(jnp.int32, (2 * EMB, 2 * EMB), 0)
    c = lax.broadcasted_iota(jnp.int32, (2 * EMB, 2 * EMB), 1)
    eye = (r == c).astype(jnp.float32)
    fcol = jnp.sum(eye * feature, axis=1, keepdims=True)
    f1 = fcol[:EMB]
    f2 = fcol[EMB:]
    rel_o[...] = jnp.dot(rel_ref[...], f1,
                         preferred_element_type=jnp.float32)
    f2_o[...] = f2


def _prep(ent_emb, history_vector, W1, b1, W2, b2, rel_table):
    return pl.pallas_call(
        _prep_body,
        out_shape=(
            jax.ShapeDtypeStruct((N_REL, 1), jnp.float32),
            jax.ShapeDtypeStruct((EMB, 1), jnp.float32),
        ),
    )(ent_emb.reshape(1, EMB), history_vector.reshape(1, EMB),
      W1, b1.reshape(1, 2 * EMB), W2, b2.reshape(1, 2 * EMB), rel_table)


def _ent_scores_body(ent_ref, f2_ref, out_ref):
    out_ref[...] = jnp.dot(ent_ref[...], f2_ref[...],
                           preferred_element_type=jnp.float32)


def _ent_scores(ent_table, f2_col):
    nb = N_ENT // ENT_BLK
    return pl.pallas_call(
        _ent_scores_body,
        grid=(nb,),
        in_specs=[
            pl.BlockSpec((ENT_BLK, EMB), lambda i: (i, 0)),
            pl.BlockSpec((EMB, 1), lambda i: (0, 0)),
        ],
        out_specs=pl.BlockSpec((ENT_BLK, 1), lambda i: (i, 0)),
        out_shape=jax.ShapeDtypeStruct((N_ENT, 1), jnp.float32),
    )(ent_table, f2_col)


def _sc_choices(rel_scores, ent_scores, rel_ids, to_ids):
    mesh = plsc.VectorSubcoreMesh(core_axis_name="c", subcore_axis_name="s")
    cp = pltpu.CompilerParams()
    if "needs_layout_passes" in pltpu.CompilerParams.__dataclass_fields__:
        cp = dataclasses.replace(cp, needs_layout_passes=False)

    @functools.partial(
        pl.kernel,
        mesh=mesh,
        compiler_params=cp,
        out_type=jax.ShapeDtypeStruct((NPAD,), jnp.float32),
        scratch_types=[
            pltpu.VMEM((REL_PAD,), jnp.float32),
            pltpu.VMEM((N_ENT,), jnp.float32),
            pltpu.VMEM((CHUNK,), jnp.int32),
            pltpu.VMEM((CHUNK,), jnp.int32),
            pltpu.VMEM((CHUNK,), jnp.float32),
        ],
    )
    def k(rel_hbm, ent_hbm, rid_hbm, tid_hbm, out_hbm,
          rel_v, ent_v, rid_v, tid_v, out_v):
        wid = lax.axis_index("s") * NC + lax.axis_index("c")
        base = wid * CHUNK
        pltpu.sync_copy(rel_hbm, rel_v)
        pltpu.sync_copy(ent_hbm, ent_v)
        pltpu.sync_copy(rid_hbm.at[pl.ds(base, CHUNK)], rid_v)
        pltpu.sync_copy(tid_hbm.at[pl.ds(base, CHUNK)], tid_v)

        @pl.loop(0, CHUNK, step=16)
        def _(i):
            ri = rid_v[pl.ds(i, 16)]
            ti = tid_v[pl.ds(i, 16)]
            rs = plsc.load_gather(rel_v, [ri])
            es = plsc.load_gather(ent_v, [ti])
            out_v[pl.ds(i, 16)] = rs + es

        pltpu.sync_copy(out_v, out_hbm.at[pl.ds(base, CHUNK)])

    return k(rel_scores, ent_scores, rel_ids, to_ids)


def _softmax_body(x_ref, o_ref):
    x = x_ref[...]
    r = lax.broadcasted_iota(jnp.int32, (NPAD // 128, 128), 0)
    c = lax.broadcasted_iota(jnp.int32, (NPAD // 128, 128), 1)
    valid = (r * 128 + c) < N_CAND
    x = jnp.where(valid, x, -jnp.inf)
    m = jnp.max(x)
    e = jnp.exp(x - m)
    s = jnp.sum(e)
    o_ref[...] = e * (1.0 / s)


def _softmax(choices_pad):
    return pl.pallas_call(
        _softmax_body,
        out_shape=jax.ShapeDtypeStruct((NPAD // 128, 128), jnp.float32),
    )(choices_pad.reshape(NPAD // 128, 128))


def kernel(ent_emb, history_vector, candidate_rel_ids, candidate_to_ids,
           rel_table, ent_table, W1, b1, W2, b2):
    rel_scores, f2_col = _prep(ent_emb, history_vector, W1, b1, W2, b2,
                               rel_table)
    ent_scores = _ent_scores(ent_table, f2_col)

    rel_s = jnp.pad(rel_scores.reshape(N_REL), (0, REL_PAD - N_REL))
    ent_s = ent_scores.reshape(N_ENT)
    rid = jnp.pad(candidate_rel_ids.astype(jnp.int32), (0, NPAD - N_CAND))
    tid = jnp.pad(candidate_to_ids.astype(jnp.int32), (0, NPAD - N_CAND))

    choices_pad = _sc_choices(rel_s, ent_s, rid, tid)
    probs = _softmax(choices_pad)
    return probs.reshape(NPAD)[:N_CAND]
